# baseline (speedup 1.0000x reference)
import functools

import jax
import jax.numpy as jnp
from jax import lax
from jax.experimental import pallas as pl
from jax.experimental.pallas import tpu as pltpu
from jax.experimental.pallas import tpu_sc as plsc

_LANES = 16


def _sc_hash_gather(n_tokens, seq, vocab, dim, n_workers, b_per_w):
    mod = vocab - 1
    mesh = plsc.VectorSubcoreMesh(core_axis_name="c", subcore_axis_name="s")
    nc = 2
    wide = 2 * dim

    @functools.partial(
        pl.kernel,
        mesh=mesh,
        out_type=(
            jax.ShapeDtypeStruct((n_tokens, wide), jnp.float32),
            jax.ShapeDtypeStruct((n_workers, b_per_w), jnp.int32),
        ),
        scratch_types=[
            pltpu.VMEM((b_per_w,), jnp.int32),
            pltpu.VMEM((b_per_w,), jnp.int32),
            pltpu.VMEM((b_per_w,), jnp.int32),
            pltpu.SMEM((b_per_w,), jnp.int32),
            pltpu.VMEM_SHARED((16, b_per_w), jnp.int32),
            pltpu.VMEM((b_per_w, wide), jnp.float32),
            pltpu.SemaphoreType.DMA,
        ],
    )
    def sc_kernel(cur_hbm, prev_hbm, table_hbm, out_hbm, idx_hbm,
                  cur_v, prev_v, idx_v, idx_s, idx_sh, rows_v, sem):
        wid = lax.axis_index("s") * nc + lax.axis_index("c")
        base = wid * b_per_w
        pltpu.sync_copy(cur_hbm.at[pl.ds(base, b_per_w)], cur_v)
        pltpu.sync_copy(prev_hbm.at[pl.ds(base, b_per_w)], prev_v)

        lane = lax.iota(jnp.int32, _LANES)
        for i in range(b_per_w // _LANES):
            c = cur_v[pl.ds(i * _LANES, _LANES)]
            p = prev_v[pl.ds(i * _LANES, _LANES)]
            h = ((c * 36313) ^ (p * 27191)) % mod
            pos = base + i * _LANES + lane
            is_first = (pos & (seq - 1)) == 0
            idx_v[pl.ds(i * _LANES, _LANES)] = jnp.where(is_first, mod, h)

        pltpu.sync_copy(idx_v, idx_hbm.at[wid])
        sid = lax.axis_index("s")
        pltpu.sync_copy(idx_v, idx_sh.at[sid])
        pltpu.sync_copy(idx_sh.at[sid], idx_s)

        def issue(i, carry):
            q = lax.shift_right_logical(idx_s[i], jnp.int32(1))
            pltpu.make_async_copy(
                table_hbm.at[pl.ds(q, 1)],
                rows_v.at[pl.ds(i, 1)],
                sem).start()
            return carry

        lax.fori_loop(jnp.int32(0), jnp.int32(b_per_w), issue, jnp.int32(0))
        pltpu.make_async_copy(
            table_hbm.at[pl.ds(jnp.int32(0), b_per_w)], rows_v, sem).wait()
        pltpu.sync_copy(rows_v, out_hbm.at[pl.ds(base, b_per_w)])

    return sc_kernel


def _make_tc_proj(seq, mod):
    def _tc_proj(cur_ref, prev_ref, rows_ref, wt_ref, scale_ref, out_ref):
        blk, wide = rows_ref.shape
        dim = wide // 2
        c = cur_ref[...]
        p = prev_ref[...]
        h = ((c * 36313) ^ (p * 27191)) % mod
        pos = (pl.program_id(0) * blk
               + lax.broadcasted_iota(jnp.int32, (blk, 1), 0))
        is_first = (pos % seq) == 0
        m = jnp.where(is_first, mod, h) & 1
        mf = m.astype(jnp.float32)
        col = lax.broadcasted_iota(jnp.int32, (blk, wide), 1)
        m_b = jnp.broadcast_to(mf, (blk, wide))
        mask = jnp.where(col >= dim, m_b, 1.0 - m_b)
        w = wt_ref[...] * scale_ref[0, 0]
        out_ref[...] = lax.dot_general(
            rows_ref[...] * mask, w, (((1,), (0,)), ((), ())),
            preferred_element_type=jnp.float32)

    return _tc_proj


def kernel(token_ids, table, W_proj, scale):
    batch, seq = token_ids.shape
    vocab, dim = table.shape
    model_dim = W_proj.shape[0]
    n = batch * seq

    tok = token_ids.astype(jnp.int32)
    cur = tok.reshape(n)
    prev = jnp.roll(tok, 1, axis=1).reshape(n)

    table2 = table.reshape(vocab // 2, 2 * dim)

    n_workers = 32
    b_per_w = n // n_workers
    rows, idx = _sc_hash_gather(n, seq, vocab, dim, n_workers, b_per_w)(
        cur, prev, table2)

    wdup = jnp.concatenate([W_proj.T, W_proj.T], axis=0)

    del idx
    blk = 512
    out = pl.pallas_call(
        _make_tc_proj(seq, vocab - 1),
        grid=(n // blk, 1),
        in_specs=[
            pl.BlockSpec((blk, 1), lambda i, j: (i, j)),
            pl.BlockSpec((blk, 1), lambda i, j: (i, j)),
            pl.BlockSpec((blk, 2 * dim), lambda i, j: (i, j)),
            pl.BlockSpec((2 * dim, model_dim), lambda i, j: (j, j)),
            pl.BlockSpec((1, 1), lambda i, j: (j, j),
                         memory_space=pltpu.SMEM),
        ],
        out_specs=pl.BlockSpec((blk, model_dim), lambda i, j: (i, j)),
        out_shape=jax.ShapeDtypeStruct((n, model_dim), jnp.float32),
    )(cur.reshape(n, 1), prev.reshape(n, 1), rows, wdup,
      scale.reshape(1, 1))

    return out.reshape(batch, seq, model_dim)
